# baseline (speedup 1.0000x reference)
import functools

import jax
import jax.numpy as jnp
from jax import lax
from jax.experimental import pallas as pl
from jax.experimental.pallas import tpu as pltpu
from jax.experimental.pallas import tpu_sc as plsc

_VOCAB_TILE = 2048


def _head_body(h_ref, wt_ref, b_ref, out_ref):
    h = h_ref[...]
    acc = b_ref[...] + h[:, 0:1] * wt_ref[0:1, :]
    acc = acc + h[:, 1:2] * wt_ref[1:2, :]
    acc = acc + h[:, 2:3] * wt_ref[2:3, :]
    acc = acc + h[:, 3:4] * wt_ref[3:4, :]
    out_ref[...] = acc


def _head(h, wt, b2):
    B = h.shape[0]
    V = wt.shape[1]
    grid = pl.cdiv(V, _VOCAB_TILE)
    return pl.pallas_call(
        _head_body,
        grid=(grid,),
        in_specs=[
            pl.BlockSpec((B, 4), lambda j: (0, 0)),
            pl.BlockSpec((4, _VOCAB_TILE), lambda j: (0, j)),
            pl.BlockSpec((1, _VOCAB_TILE), lambda j: (0, j)),
        ],
        out_specs=pl.BlockSpec((B, _VOCAB_TILE), lambda j: (0, j)),
        out_shape=jax.ShapeDtypeStruct((B, V), jnp.float32),
    )(h, wt, b2)


_LANES = 16


def _sc_gather(emb, ids):
    B = ids.shape[0]
    D = emb.shape[1]
    rows_per_blk = 128 // D
    emb3 = emb.reshape(-1, 128)
    info = plsc.get_sparse_core_info()
    nw = info.num_cores * info.num_subcores
    bw = B // nw
    mesh = plsc.VectorSubcoreMesh(core_axis_name="c", subcore_axis_name="s")

    @functools.partial(
        pl.kernel,
        mesh=mesh,
        out_type=jax.ShapeDtypeStruct((B * D,), jnp.float32),
        scratch_types=[
            pltpu.VMEM((bw,), jnp.int32),
            pltpu.VMEM((bw,), jnp.int32),
            pltpu.VMEM((bw, 128), jnp.float32),
            pltpu.VMEM((bw * D,), jnp.float32),
            pltpu.SemaphoreType.DMA,
        ],
        compiler_params=pltpu.CompilerParams(needs_layout_passes=False),
    )
    def k(emb_hbm, ids_hbm, out_hbm, idx_v, blk_v, rows_v, out_v, sem):
        wid = lax.axis_index("s") * info.num_cores + lax.axis_index("c")
        base = wid * bw
        pltpu.sync_copy(ids_hbm.at[pl.ds(base, bw)], idx_v)
        for c in range(bw // _LANES):
            chunk = idx_v[pl.ds(c * _LANES, _LANES)]
            blk_v[pl.ds(c * _LANES, _LANES)] = lax.shift_right_logical(chunk, 5)
        pltpu.async_copy(emb_hbm.at[blk_v], rows_v, sem).wait()
        for g in range(bw * D // _LANES):
            n = lax.iota(jnp.int32, _LANES) + g * _LANES
            tok = lax.shift_right_logical(n, 2)
            tid = plsc.load_gather(idx_v, [tok])
            col = lax.shift_left(
                lax.bitwise_and(tid, rows_per_blk - 1), 2
            ) + lax.bitwise_and(n, D - 1)
            out_v[pl.ds(g * _LANES, _LANES)] = plsc.load_gather(
                rows_v, [tok, col]
            )
        pltpu.sync_copy(out_v, out_hbm.at[pl.ds(base * D, bw * D)])

    return k(emb3, ids)


def kernel(input_ids, emb, W, b):
    ids = input_ids.astype(jnp.int32)
    h = _sc_gather(emb, ids).reshape(ids.shape[0], emb.shape[1])
    wt = W.T
    b2 = b.reshape(1, -1)
    return _head(h, wt, b2)
